# baseline (device time: 86315 ns/iter reference)
import jax
import jax.numpy as jnp
from jax import lax
from jax.experimental import pallas as pl
from jax.experimental.pallas import tpu as pltpu

T = 1024
Q = 512
D = 1024
F = 2048
E_LOC = 2

_MESH = pl.DeviceIdType.MESH


def kernel(x, assign, W1, W2):
    assign2 = assign.reshape(T, 1)

    def body(x_ref, a_ref, w1_ref, w2_ref, out_ref,
             qx_ref, qa_ref, pm_ref, precv_ref, qf_ref, outb_ref,
             send_sems, recv_sems):
        my_x = lax.axis_index("x")
        my_y = lax.axis_index("y")
        my_z = lax.axis_index("z")
        partner = (1 - my_x, my_y, my_z)
        z_sib = (my_x, my_y, 1 - my_z)
        y_sib = (my_x, 1 - my_y, my_z)
        d_sib = (my_x, 1 - my_y, 1 - my_z)

        is_owner = my_y == my_x
        rows = pl.ds(my_z * Q, Q)

        barrier = pltpu.get_barrier_semaphore()
        for nbr in (partner, z_sib, y_sib, d_sib):
            pl.semaphore_signal(barrier, inc=1, device_id=nbr,
                                device_id_type=_MESH)
        pl.semaphore_wait(barrier, 4)

        def p1_rdmas():
            r1 = pltpu.make_async_remote_copy(
                src_ref=qx_ref, dst_ref=qx_ref,
                send_sem=send_sems.at[0], recv_sem=recv_sems.at[0],
                device_id=partner, device_id_type=_MESH)
            r2 = pltpu.make_async_remote_copy(
                src_ref=qa_ref, dst_ref=qa_ref,
                send_sem=send_sems.at[1], recv_sem=recv_sems.at[1],
                device_id=partner, device_id_type=_MESH)
            return r1, r2

        @pl.when(is_owner)
        def _():
            qx_ref[...] = x_ref[rows, :].astype(jnp.bfloat16)
            qa_ref[...] = a_ref[rows, :]
            r1, r2 = p1_rdmas()
            r1.start()
            r2.start()

        @pl.when(jnp.logical_not(is_owner))
        def _():
            r1, r2 = p1_rdmas()
            r1.wait_recv()
            r2.wait_recv()

        acc = None
        for k in range(E_LOC):
            ge = E_LOC * my_x + k
            xm = jnp.where(qa_ref[...] == ge, qx_ref[...], jnp.bfloat16(0.0))
            h = jnp.dot(xm, w1_ref[k].astype(jnp.bfloat16),
                        preferred_element_type=jnp.float32)
            h = jnp.maximum(h, 0.0).astype(jnp.bfloat16)
            o = jnp.dot(h, w2_ref[k].astype(jnp.bfloat16),
                        preferred_element_type=jnp.float32)
            acc = o if acc is None else acc + o
        pm_ref[...] = acc.astype(jnp.bfloat16)

        @pl.when(is_owner)
        def _():
            r1, r2 = p1_rdmas()
            r1.wait_send()
            r2.wait_send()

        r3 = pltpu.make_async_remote_copy(
            src_ref=pm_ref, dst_ref=precv_ref,
            send_sem=send_sems.at[2], recv_sem=recv_sems.at[2],
            device_id=partner, device_id_type=_MESH)
        r3.start()
        r3.wait()
        qf_ref[...] = pm_ref[...] + precv_ref[...]

        is_holder = is_owner

        def p4_desc(target, slot, si):
            return pltpu.make_async_remote_copy(
                src_ref=qf_ref, dst_ref=outb_ref.at[slot],
                send_sem=send_sems.at[si], recv_sem=recv_sems.at[si],
                device_id=target, device_id_type=_MESH)

        @pl.when(jnp.logical_and(is_holder, my_z == 0))
        def _():
            outb_ref[0] = qf_ref[...]

        @pl.when(jnp.logical_and(is_holder, my_z == 1))
        def _():
            outb_ref[1] = qf_ref[...]

        @pl.when(is_holder)
        def _():
            ra = p4_desc(z_sib, my_z, 3)
            rb = p4_desc(y_sib, my_z, 4)
            rc = p4_desc(d_sib, my_z, 5)
            ra.start()
            rb.start()
            rc.start()
            p4_desc(z_sib, 1 - my_z, 3).wait_recv()
            ra.wait_send()
            rb.wait_send()
            rc.wait_send()

        @pl.when(jnp.logical_not(is_holder))
        def _():
            p4_desc(y_sib, my_z, 4).wait_recv()
            p4_desc(d_sib, 1 - my_z, 5).wait_recv()

        out_ref[0:Q, :] = outb_ref[0].astype(jnp.float32)
        out_ref[Q:T, :] = outb_ref[1].astype(jnp.float32)

    return pl.pallas_call(
        body,
        out_shape=jax.ShapeDtypeStruct((T, D), jnp.float32),
        in_specs=[pl.BlockSpec(memory_space=pltpu.VMEM)] * 4,
        out_specs=pl.BlockSpec(memory_space=pltpu.VMEM),
        scratch_shapes=[
            pltpu.VMEM((Q, D), jnp.bfloat16),
            pltpu.VMEM((Q, 1), jnp.int32),
            pltpu.VMEM((Q, D), jnp.bfloat16),
            pltpu.VMEM((Q, D), jnp.bfloat16),
            pltpu.VMEM((Q, D), jnp.bfloat16),
            pltpu.VMEM((2, Q, D), jnp.bfloat16),
            pltpu.SemaphoreType.DMA((6,)),
            pltpu.SemaphoreType.DMA((6,)),
        ],
        compiler_params=pltpu.CompilerParams(
            collective_id=0, vmem_limit_bytes=110 * 1024 * 1024),
    )(x, assign2, W1, W2)


# device time: 29332 ns/iter; 2.9427x vs baseline; 2.9427x over previous
import jax
import jax.numpy as jnp
from jax import lax
from jax.experimental import pallas as pl
from jax.experimental.pallas import tpu as pltpu

T = 1024
Q = 512
D = 1024
A = 128
F = 2048
E_LOC = 2
NC = 4
CH = Q // NC

_MESH = pl.DeviceIdType.MESH

S_P1, S_P3, S_Z, S_Y, S_R = 0, NC, 2 * NC, 3 * NC, 4 * NC


def kernel(x, assign, W1, W2):
    xa = jnp.concatenate(
        [x, jnp.broadcast_to(assign.astype(jnp.float32)[:, None], (T, A))],
        axis=1)

    def body(xa_ref, w1_ref, w2_ref, out_ref,
             qxa_ref, pm_ref, precv_ref, qf_ref, yrecv_ref, zrecv_ref,
             send_sems, recv_sems):
        my_x = lax.axis_index("x")
        my_y = lax.axis_index("y")
        my_z = lax.axis_index("z")
        partner = (1 - my_x, my_y, my_z)
        z_sib = (my_x, my_y, 1 - my_z)
        y_sib = (my_x, 1 - my_y, my_z)

        is_owner = my_y == my_x
        not_owner = jnp.logical_not(is_owner)

        def chk(c):
            return pl.ds(c * CH, CH)

        def dcopy(src, dst, sem_i, target):
            return pltpu.make_async_remote_copy(
                src_ref=src, dst_ref=dst,
                send_sem=send_sems.at[sem_i], recv_sem=recv_sems.at[sem_i],
                device_id=target, device_id_type=_MESH)

        def p1_d(c):
            return dcopy(qxa_ref.at[chk(c), :], qxa_ref.at[chk(c), :],
                         S_P1 + c, partner)

        def p3_d(c):
            return dcopy(pm_ref.at[chk(c), :], precv_ref.at[chk(c), :],
                         S_P3 + c, partner)

        def z_d(c):
            return dcopy(qf_ref.at[chk(c), :], zrecv_ref.at[chk(c), :],
                         S_Z + c, z_sib)

        def y_d(c):
            return dcopy(qf_ref.at[chk(c), :], yrecv_ref.at[chk(c), :],
                         S_Y + c, y_sib)

        def r_d(c):
            return dcopy(yrecv_ref.at[chk(c), :], zrecv_ref.at[chk(c), :],
                         S_R + c, z_sib)

        barrier = pltpu.get_barrier_semaphore()
        for nbr in (partner, z_sib, y_sib):
            pl.semaphore_signal(barrier, inc=1, device_id=nbr,
                                device_id_type=_MESH)
        pl.semaphore_wait(barrier, 3)

        @pl.when(is_owner)
        def _():
            for c in range(NC):
                src = pl.ds(my_z * Q + c * CH, CH)
                qxa_ref[chk(c), :] = xa_ref[src, :].astype(jnp.bfloat16)
                p1_d(c).start()

        w1b = [w1_ref[k].astype(jnp.bfloat16) for k in range(E_LOC)]
        w2b = [w2_ref[k].astype(jnp.bfloat16) for k in range(E_LOC)]

        for c in range(NC):
            @pl.when(not_owner)
            def _(c=c):
                p1_d(c).wait_recv()

            tile = qxa_ref[chk(c), :]
            av = tile[:, D:D + 1]
            acc = None
            for k in range(E_LOC):
                ge = (E_LOC * my_x + k).astype(jnp.bfloat16)
                xm = jnp.where(av == ge, tile[:, 0:D], jnp.bfloat16(0.0))
                h = jnp.dot(xm, w1b[k], preferred_element_type=jnp.float32)
                h = jnp.maximum(h, 0.0).astype(jnp.bfloat16)
                o = jnp.dot(h, w2b[k], preferred_element_type=jnp.float32)
                acc = o if acc is None else acc + o
            pm_ref[chk(c), :] = acc.astype(jnp.bfloat16)

            @pl.when(not_owner)
            def _(c=c):
                p3_d(c).start()

        @pl.when(is_owner)
        def _():
            for c in range(NC):
                p1_d(c).wait_send()
            for c in range(NC):
                p3_d(c).wait_recv()
                qf_ref[chk(c), :] = pm_ref[chk(c), :] + precv_ref[chk(c), :]
                z_d(c).start()
                y_d(c).start()
            for c in range(NC):
                z_d(c).wait_recv()
            for c in range(NC):
                z_d(c).wait_send()
                y_d(c).wait_send()

        @pl.when(not_owner)
        def _():
            for c in range(NC):
                y_d(c).wait_recv()
                r_d(c).start()
            for c in range(NC):
                r_d(c).wait_recv()
            for c in range(NC):
                p3_d(c).wait_send()
                r_d(c).wait_send()

        own = jnp.where(is_owner, qf_ref[...], yrecv_ref[...])
        other = zrecv_ref[...]
        z0 = my_z == 0
        out_ref[0:Q, :] = jnp.where(z0, own, other).astype(jnp.float32)
        out_ref[Q:T, :] = jnp.where(z0, other, own).astype(jnp.float32)

    return pl.pallas_call(
        body,
        out_shape=jax.ShapeDtypeStruct((T, D), jnp.float32),
        in_specs=[pl.BlockSpec(memory_space=pltpu.VMEM)] * 3,
        out_specs=pl.BlockSpec(memory_space=pltpu.VMEM),
        scratch_shapes=[
            pltpu.VMEM((Q, D + A), jnp.bfloat16),
            pltpu.VMEM((Q, D), jnp.bfloat16),
            pltpu.VMEM((Q, D), jnp.bfloat16),
            pltpu.VMEM((Q, D), jnp.bfloat16),
            pltpu.VMEM((Q, D), jnp.bfloat16),
            pltpu.VMEM((Q, D), jnp.bfloat16),
            pltpu.SemaphoreType.DMA((5 * NC,)),
            pltpu.SemaphoreType.DMA((5 * NC,)),
        ],
        compiler_params=pltpu.CompilerParams(
            collective_id=0, vmem_limit_bytes=110 * 1024 * 1024),
    )(xa, W1, W2)
